# Initial kernel scaffold; baseline (speedup 1.0000x reference)
import jax
import jax.numpy as jnp
from jax.experimental import pallas as pl


def kernel(embeddings, target, triplets):
    raise NotImplementedError("write your pallas kernel here")


# speedup vs baseline: 1.5781x; 1.5781x over previous
import functools

import jax
import jax.numpy as jnp
from jax import lax
from jax.experimental import pallas as pl
from jax.experimental.pallas import tpu as pltpu
from jax.experimental.pallas import tpu_sc as plsc

_MARGIN = 0.2
_EPS = 1e-12

_L = 16
_NC, _NS = 2, 16
_NW = _NC * _NS
_N_TRIP = 32768
_D = 128
_T_W = _N_TRIP // _NW
_C = 128
_NCH = _T_W // _C
_NG = _C // _L


def _sqrt16(x):
    i = plsc.bitcast(x, jnp.int32)
    i = jnp.int32(0x5F3759DF) - (i >> 1)
    y = plsc.bitcast(i, jnp.float32)
    xh = x * jnp.float32(0.5)
    for _ in range(3):
        y = y * (jnp.float32(1.5) - xh * y * y)
    return x * y


def _body(emb, ia, ip, inn, ap_out, an_out, part_out,
          idx_a, idx_p, idx_n, ra, rp, rn, apv, anv, lossv, sem):
    wid = lax.axis_index("s") * _NC + lax.axis_index("c")
    lane = lax.iota(jnp.int32, _L)
    loss_acc = jnp.zeros((_L,), jnp.float32)

    for ch in range(_NCH):
        base = pl.multiple_of(wid * _T_W + ch * _C, _C)
        pltpu.sync_copy(ia.at[pl.ds(base, _C)], idx_a)
        pltpu.sync_copy(ip.at[pl.ds(base, _C)], idx_p)
        pltpu.sync_copy(inn.at[pl.ds(base, _C)], idx_n)
        ca = pltpu.async_copy(emb.at[idx_a], ra, sem)
        cp = pltpu.async_copy(emb.at[idx_p], rp, sem)
        cn = pltpu.async_copy(emb.at[idx_n], rn, sem)
        ca.wait()
        cp.wait()
        cn.wait()

        def grp_body(g, acc, _ch=ch):
            row = g * _L + lane
            acc_ap = jnp.zeros((_L,), jnp.float32)
            acc_an = jnp.zeros((_L,), jnp.float32)
            for d in range(_D):
                col = jnp.full((_L,), d, jnp.int32)
                va = plsc.load_gather(ra, [row, col])
                vp = plsc.load_gather(rp, [row, col])
                vn = plsc.load_gather(rn, [row, col])
                tp = va - vp + jnp.float32(_EPS)
                tn = va - vn + jnp.float32(_EPS)
                acc_ap = acc_ap + tp * tp
                acc_an = acc_an + tn * tn
            d_ap = _sqrt16(acc_ap)
            d_an = _sqrt16(acc_an)
            off = _ch * _C + g * _L
            apv[pl.ds(off, _L)] = d_ap
            anv[pl.ds(off, _L)] = d_an
            return acc + jnp.maximum(d_ap - d_an + jnp.float32(_MARGIN),
                                     jnp.float32(0.0))

        loss_acc = lax.fori_loop(0, _NG, grp_body, loss_acc)

    lossv[...] = loss_acc
    out_base = pl.multiple_of(wid * _T_W, _T_W)
    pltpu.sync_copy(apv, ap_out.at[pl.ds(out_base, _T_W)])
    pltpu.sync_copy(anv, an_out.at[pl.ds(out_base, _T_W)])
    pltpu.sync_copy(lossv, part_out.at[wid])


_triplet_sc = functools.partial(
    pl.kernel,
    out_type=[
        jax.ShapeDtypeStruct((_N_TRIP,), jnp.float32),
        jax.ShapeDtypeStruct((_N_TRIP,), jnp.float32),
        jax.ShapeDtypeStruct((_NW, _L), jnp.float32),
    ],
    mesh=plsc.VectorSubcoreMesh(core_axis_name="c", subcore_axis_name="s"),
    compiler_params=pltpu.CompilerParams(needs_layout_passes=False),
    scratch_types=[
        pltpu.VMEM((_C,), jnp.int32),
        pltpu.VMEM((_C,), jnp.int32),
        pltpu.VMEM((_C,), jnp.int32),
        pltpu.VMEM((_C, _D), jnp.float32),
        pltpu.VMEM((_C, _D), jnp.float32),
        pltpu.VMEM((_C, _D), jnp.float32),
        pltpu.VMEM((_T_W,), jnp.float32),
        pltpu.VMEM((_T_W,), jnp.float32),
        pltpu.VMEM((_L,), jnp.float32),
        pltpu.SemaphoreType.DMA,
    ],
)(_body)


def kernel(embeddings, target, triplets):
    del target
    ia = triplets[:, 0]
    ip = triplets[:, 1]
    inn = triplets[:, 2]
    ap, an, part = _triplet_sc(embeddings, ia, ip, inn)
    loss = jnp.sum(part) / jnp.float32(_N_TRIP)
    tdist = jnp.concatenate([ap, an], axis=0)
    ttgt = jnp.concatenate(
        [jnp.ones((_N_TRIP,), jnp.float32),
         jnp.zeros((_N_TRIP,), jnp.float32)], axis=0)
    return (loss, ap, an, tdist, ttgt)


# speedup vs baseline: 4.3967x; 2.7860x over previous
import functools

import jax
import jax.numpy as jnp
from jax import lax
from jax.experimental import pallas as pl
from jax.experimental.pallas import tpu as pltpu
from jax.experimental.pallas import tpu_sc as plsc

_MARGIN = 0.2
_EPS = 1e-12

_L = 16
_NC, _NS = 2, 16
_NW = _NC * _NS
_N_TRIP = 32768
_D = 128
_T_W = _N_TRIP // _NW
_C = 128
_NCH = _T_W // _C
_NG = _C // _L


def _sqrt16(x):
    i = plsc.bitcast(x, jnp.int32)
    i = jnp.int32(0x5F3759DF) - (i >> 1)
    y = plsc.bitcast(i, jnp.float32)
    xh = x * jnp.float32(0.5)
    for _ in range(3):
        y = y * (jnp.float32(1.5) - xh * y * y)
    return x * y


def _body(emb, ia, ip, inn, ap_out, an_out, part_out,
          idx_a, idx_p, idx_n, ra, rp, rn, apv, anv, lossv, sem):
    wid = lax.axis_index("s") * _NC + lax.axis_index("c")
    lane = lax.iota(jnp.int32, _L)
    loss_acc = jnp.zeros((_L,), jnp.float32)

    for ch in range(_NCH):
        base = pl.multiple_of(wid * _T_W + ch * _C, _C)
        pltpu.sync_copy(ia.at[pl.ds(base, _C)], idx_a)
        pltpu.sync_copy(ip.at[pl.ds(base, _C)], idx_p)
        pltpu.sync_copy(inn.at[pl.ds(base, _C)], idx_n)
        ca = pltpu.async_copy(emb.at[idx_a], ra, sem)
        cp = pltpu.async_copy(emb.at[idx_p], rp, sem)
        cn = pltpu.async_copy(emb.at[idx_n], rn, sem)
        ca.wait()
        cp.wait()
        cn.wait()

        def grp_body(g, acc, _ch=ch):
            base_r = g * _L
            acc_ap = jnp.zeros((_L,), jnp.float32)
            acc_an = jnp.zeros((_L,), jnp.float32)
            for rs in range(_L):
                r = base_r + rs
                p0 = jnp.zeros((_L,), jnp.float32)
                p1 = jnp.zeros((_L,), jnp.float32)
                n0 = jnp.zeros((_L,), jnp.float32)
                n1 = jnp.zeros((_L,), jnp.float32)
                for s_ in range(_D // _L):
                    sl = pl.ds(s_ * _L, _L)
                    va = ra[r, sl]
                    vp = rp[r, sl]
                    vn = rn[r, sl]
                    tp = va - vp + jnp.float32(_EPS)
                    tn = va - vn + jnp.float32(_EPS)
                    if s_ % 2 == 0:
                        p0 = p0 + tp * tp
                        n0 = n0 + tn * tn
                    else:
                        p1 = p1 + tp * tp
                        n1 = n1 + tn * tn
                sap = jnp.sum(p0 + p1)
                san = jnp.sum(n0 + n1)
                m = lane == rs
                acc_ap = jnp.where(m, sap, acc_ap)
                acc_an = jnp.where(m, san, acc_an)
            d_ap = _sqrt16(acc_ap)
            d_an = _sqrt16(acc_an)
            off = _ch * _C + g * _L
            apv[pl.ds(off, _L)] = d_ap
            anv[pl.ds(off, _L)] = d_an
            return acc + jnp.maximum(d_ap - d_an + jnp.float32(_MARGIN),
                                     jnp.float32(0.0))

        loss_acc = lax.fori_loop(0, _NG, grp_body, loss_acc)

    lossv[...] = loss_acc
    out_base = pl.multiple_of(wid * _T_W, _T_W)
    pltpu.sync_copy(apv, ap_out.at[pl.ds(out_base, _T_W)])
    pltpu.sync_copy(anv, an_out.at[pl.ds(out_base, _T_W)])
    pltpu.sync_copy(lossv, part_out.at[wid])


_triplet_sc = functools.partial(
    pl.kernel,
    out_type=[
        jax.ShapeDtypeStruct((_N_TRIP,), jnp.float32),
        jax.ShapeDtypeStruct((_N_TRIP,), jnp.float32),
        jax.ShapeDtypeStruct((_NW, _L), jnp.float32),
    ],
    mesh=plsc.VectorSubcoreMesh(core_axis_name="c", subcore_axis_name="s"),
    compiler_params=pltpu.CompilerParams(needs_layout_passes=False),
    scratch_types=[
        pltpu.VMEM((_C,), jnp.int32),
        pltpu.VMEM((_C,), jnp.int32),
        pltpu.VMEM((_C,), jnp.int32),
        pltpu.VMEM((_C, _D), jnp.float32),
        pltpu.VMEM((_C, _D), jnp.float32),
        pltpu.VMEM((_C, _D), jnp.float32),
        pltpu.VMEM((_T_W,), jnp.float32),
        pltpu.VMEM((_T_W,), jnp.float32),
        pltpu.VMEM((_L,), jnp.float32),
        pltpu.SemaphoreType.DMA,
    ],
)(_body)


def kernel(embeddings, target, triplets):
    del target
    ia = triplets[:, 0]
    ip = triplets[:, 1]
    inn = triplets[:, 2]
    ap, an, part = _triplet_sc(embeddings, ia, ip, inn)
    loss = jnp.sum(part) / jnp.float32(_N_TRIP)
    tdist = jnp.concatenate([ap, an], axis=0)
    ttgt = jnp.concatenate(
        [jnp.ones((_N_TRIP,), jnp.float32),
         jnp.zeros((_N_TRIP,), jnp.float32)], axis=0)
    return (loss, ap, an, tdist, ttgt)


# speedup vs baseline: 5.5068x; 1.2525x over previous
import functools

import jax
import jax.numpy as jnp
from jax import lax
from jax.experimental import pallas as pl
from jax.experimental.pallas import tpu as pltpu
from jax.experimental.pallas import tpu_sc as plsc

_MARGIN = 0.2
_ZDIST = 1.13137085e-11

_L = 16
_NC, _NS = 2, 16
_NW = _NC * _NS
_N_TRIP = 32768
_D = 128
_T_W = _N_TRIP // _NW
_C = 128
_NCH = _T_W // _C
_NG = _C // _L
_PB = _L + 1


def _sqrt16(x):
    i = plsc.bitcast(x, jnp.int32)
    i = jnp.int32(0x5F3759DF) - (i >> 1)
    y = plsc.bitcast(i, jnp.float32)
    xh = x * jnp.float32(0.5)
    for _ in range(3):
        y = y * (jnp.float32(1.5) - xh * y * y)
    return x * y


def _body(emb, ia, ip, inn, ap_out, an_out, part_out,
          idxa, idxp, idxn, ra0, rp0, rn0, ra1, rp1, rn1,
          apv, anv, pbuf, nbuf, lossv, sem0, sem1):
    wid = lax.axis_index("s") * _NC + lax.axis_index("c")
    lane = lax.iota(jnp.int32, _L)
    base_t = pl.multiple_of(wid * _T_W, _T_W)

    pltpu.sync_copy(ia.at[pl.ds(base_t, _T_W)], idxa)
    pltpu.sync_copy(ip.at[pl.ds(base_t, _T_W)], idxp)
    pltpu.sync_copy(inn.at[pl.ds(base_t, _T_W)], idxn)

    bufs = ((ra0, rp0, rn0), (ra1, rp1, rn1))
    sems = (sem0, sem1)

    def fire(ch):
        b = bufs[ch % 2]
        s = sems[ch % 2]
        sl = pl.ds(ch * _C, _C)
        return (pltpu.async_copy(emb.at[idxa.at[sl]], b[0], s),
                pltpu.async_copy(emb.at[idxp.at[sl]], b[1], s),
                pltpu.async_copy(emb.at[idxn.at[sl]], b[2], s))

    pend = fire(0)
    loss_acc = jnp.zeros((_L,), jnp.float32)

    for ch in range(_NCH):
        for c in pend:
            c.wait()
        if ch + 1 < _NCH:
            pend = fire(ch + 1)
        ra, rp, rn = bufs[ch % 2]

        def grp_body(g, acc, _ch=ch, ra=ra, rp=rp, rn=rn):
            base_r = g * _L
            for rs in range(_L):
                r = base_r + rs
                p0 = jnp.zeros((_L,), jnp.float32)
                p1 = jnp.zeros((_L,), jnp.float32)
                n0 = jnp.zeros((_L,), jnp.float32)
                n1 = jnp.zeros((_L,), jnp.float32)
                for s_ in range(_D // _L):
                    sl = pl.ds(s_ * _L, _L)
                    va = ra[r, sl]
                    vp = rp[r, sl]
                    vn = rn[r, sl]
                    tp = va - vp
                    tn = va - vn
                    if s_ % 2 == 0:
                        p0 = p0 + tp * tp
                        n0 = n0 + tn * tn
                    else:
                        p1 = p1 + tp * tp
                        n1 = n1 + tn * tn
                pbuf[pl.ds(rs * _PB, _L)] = p0 + p1
                nbuf[pl.ds(rs * _PB, _L)] = n0 + n1
            d2p = jnp.zeros((_L,), jnp.float32)
            d2n = jnp.zeros((_L,), jnp.float32)
            for c_ in range(_L):
                d2p = d2p + plsc.load_gather(pbuf, [lane * _PB + c_])
                d2n = d2n + plsc.load_gather(nbuf, [lane * _PB + c_])
            zero = jnp.float32(0.0)
            zd = jnp.float32(_ZDIST)
            d_ap = jnp.where(d2p == zero, zd, _sqrt16(d2p))
            d_an = jnp.where(d2n == zero, zd, _sqrt16(d2n))
            off = _ch * _C + g * _L
            apv[pl.ds(off, _L)] = d_ap
            anv[pl.ds(off, _L)] = d_an
            return acc + jnp.maximum(d_ap - d_an + jnp.float32(_MARGIN), zero)

        loss_acc = lax.fori_loop(0, _NG, grp_body, loss_acc)

    lossv[...] = loss_acc
    pltpu.sync_copy(apv, ap_out.at[pl.ds(base_t, _T_W)])
    pltpu.sync_copy(anv, an_out.at[pl.ds(base_t, _T_W)])
    pltpu.sync_copy(lossv, part_out.at[wid])


_triplet_sc = functools.partial(
    pl.kernel,
    out_type=[
        jax.ShapeDtypeStruct((_N_TRIP,), jnp.float32),
        jax.ShapeDtypeStruct((_N_TRIP,), jnp.float32),
        jax.ShapeDtypeStruct((_NW, _L), jnp.float32),
    ],
    mesh=plsc.VectorSubcoreMesh(core_axis_name="c", subcore_axis_name="s"),
    compiler_params=pltpu.CompilerParams(needs_layout_passes=False),
    scratch_types=[
        pltpu.VMEM((_T_W,), jnp.int32),
        pltpu.VMEM((_T_W,), jnp.int32),
        pltpu.VMEM((_T_W,), jnp.int32),
        pltpu.VMEM((_C, _D), jnp.float32),
        pltpu.VMEM((_C, _D), jnp.float32),
        pltpu.VMEM((_C, _D), jnp.float32),
        pltpu.VMEM((_C, _D), jnp.float32),
        pltpu.VMEM((_C, _D), jnp.float32),
        pltpu.VMEM((_C, _D), jnp.float32),
        pltpu.VMEM((_T_W,), jnp.float32),
        pltpu.VMEM((_T_W,), jnp.float32),
        pltpu.VMEM((_L * _PB,), jnp.float32),
        pltpu.VMEM((_L * _PB,), jnp.float32),
        pltpu.VMEM((_L,), jnp.float32),
        pltpu.SemaphoreType.DMA,
        pltpu.SemaphoreType.DMA,
    ],
)(_body)


def kernel(embeddings, target, triplets):
    del target
    ia = triplets[:, 0]
    ip = triplets[:, 1]
    inn = triplets[:, 2]
    ap, an, part = _triplet_sc(embeddings, ia, ip, inn)
    loss = jnp.sum(part) / jnp.float32(_N_TRIP)
    tdist = jnp.concatenate([ap, an], axis=0)
    ttgt = jnp.concatenate(
        [jnp.ones((_N_TRIP,), jnp.float32),
         jnp.zeros((_N_TRIP,), jnp.float32)], axis=0)
    return (loss, ap, an, tdist, ttgt)
